# Initial kernel scaffold; baseline (speedup 1.0000x reference)
import jax
import jax.numpy as jnp
from jax.experimental import pallas as pl


def kernel(x, edge_index, edge_weight, edge_label_index, explicit_weight, W1rel, b1, W1root, W2rel, b2, W2root, Wd1, bd1, Wd2, bd2):
    raise NotImplementedError("write your pallas kernel here")


# speedup vs baseline: 4.9621x; 4.9621x over previous
import dataclasses
import functools

import jax
import jax.numpy as jnp
from jax import lax
from jax.experimental import pallas as pl
from jax.experimental.pallas import tpu as pltpu
from jax.experimental.pallas import tpu_sc as plsc

_NC = 2
_NS = 16
_NW = _NC * _NS
_L = 16
_NPAD = 10240
_DBG_NO_LOOP = True
_DBG_NO_SPMEM_READ = True
_DBG_NO_HBM_WRITE = True
_DBG_NO_CNT = True


def _vector_mesh():
    return plsc.VectorSubcoreMesh(core_axis_name="c", subcore_axis_name="s")


def _sc_params(tc_tiling=True):
    cp = pltpu.CompilerParams()
    if "needs_layout_passes" in pltpu.CompilerParams.__dataclass_fields__:
        cp = dataclasses.replace(cp, needs_layout_passes=False)
    if not tc_tiling:
        cp = dataclasses.replace(cp, use_tc_tiling_on_sc=False)
    return cp


def _segsum_sc(x, src, dst, w, with_cnt):
    n, feat = x.shape
    (e,) = src.shape
    ept = e // _NW
    chunk = 80
    nsup = 5
    sub = ept // nsup
    nchunk = sub // chunk
    npad = _NPAD
    npart = npad // _NS

    outs = [jax.ShapeDtypeStruct((_NC * npad, feat), jnp.float32)]
    if with_cnt:
        outs.append(jax.ShapeDtypeStruct((_NC * npad, feat), jnp.float32))
    scr = [
        pltpu.VMEM((sub,), jnp.int32),
        pltpu.VMEM((chunk,), jnp.int32),
        pltpu.VMEM((sub,), jnp.float32),
        pltpu.VMEM((chunk, feat), jnp.float32),
        pltpu.VMEM((chunk, feat), jnp.float32),
        pltpu.VMEM_SHARED((npad, feat), jnp.float32),
        pltpu.SemaphoreType.DMA,
    ]

    def body(refs):
        if with_cnt:
            (x_hbm, src_hbm, dst_hbm, w_hbm, acc_out, cnt_out,
             sidx_v, didx_v, w_v, rows_v, hop_v, acc_sh, sem) = refs
        else:
            (x_hbm, src_hbm, dst_hbm, w_hbm, acc_out,
             sidx_v, didx_v, w_v, rows_v, hop_v, acc_sh, sem) = refs
        c = lax.axis_index("c")
        s = lax.axis_index("s")
        wid = c * _NS + s

        zv = jnp.zeros((_L,), jnp.float32)
        ov = zv + 1.0

        def fill(buf, vec):
            for j in range(chunk):
                row = buf.at[j]
                for k in range(feat // _L):
                    row[pl.ds(k * _L, _L)] = vec

        def zero_acc():
            fill(hop_v, zv)
            for t in range(npart // chunk):
                pltpu.sync_copy(
                    hop_v, acc_sh.at[pl.ds(s * npart + t * chunk, chunk)])

        def write_acc(out_hbm):
            for t in range(npart // chunk):
                r0 = s * npart + t * chunk
                pltpu.sync_copy(acc_sh.at[pl.ds(r0, chunk)], hop_v)
                pltpu.sync_copy(hop_v,
                                out_hbm.at[pl.ds(c * npad + r0, chunk)])

        zero_acc()
        plsc.subcore_barrier()

        @pl.loop(0, nsup)
        def _(sc):
            pltpu.sync_copy(src_hbm.at[pl.ds(wid * ept + sc * sub, sub)],
                            sidx_v)
            pltpu.sync_copy(w_hbm.at[pl.ds(wid * ept + sc * sub, sub)], w_v)

            @pl.loop(0, nchunk)
            def _(i):
                pltpu.sync_copy(
                    dst_hbm.at[pl.ds(wid * ept + sc * sub + i * chunk, chunk)],
                    didx_v)
                pltpu.async_copy(x_hbm.at[sidx_v.at[pl.ds(i * chunk, chunk)]],
                                 rows_v, sem).wait()

                for g in range(chunk // _L):
                    wrow = w_v[pl.ds(i * chunk + g * _L, _L)]
                    for j in range(_L):
                        ws = jnp.full((_L,), wrow[j], jnp.float32)
                        row = rows_v.at[g * _L + j]
                        for k in range(feat // _L):
                            sl = pl.ds(k * _L, _L)
                            row[sl] = row[sl] * ws

                pltpu.sync_copy(rows_v, acc_sh.at[didx_v], add=True)

        plsc.subcore_barrier()
        write_acc(acc_out)

        if with_cnt:
            plsc.subcore_barrier()
            zero_acc()
            fill(rows_v, ov)
            plsc.subcore_barrier()

            @pl.loop(0, ept // chunk)
            def _(i):
                pltpu.sync_copy(
                    dst_hbm.at[pl.ds(wid * ept + i * chunk, chunk)], didx_v)
                pltpu.sync_copy(rows_v, acc_sh.at[didx_v], add=True)

            plsc.subcore_barrier()
            write_acc(cnt_out)

    if with_cnt:
        @functools.partial(pl.kernel, out_type=tuple(outs),
                           mesh=_vector_mesh(), scratch_types=scr)
        def k(*refs):
            body(refs)

        o1, o2 = k(x, src, dst, w)
        return (o1.reshape(_NC, npad, feat), o2.reshape(_NC, npad, feat))
    else:

        @functools.partial(pl.kernel, out_type=outs[0],
                           mesh=_vector_mesh(), scratch_types=scr)
        def k(*refs):
            body(refs)

        return k(x, src, dst, w).reshape(_NC, npad, feat)


def _tc_layer1(sp, cntp, x, wrel, brel, wroot):
    n, feat = x.shape
    bn = 1000

    def body(sp_ref, cnt_ref, x_ref, wrel_ref, b_ref, wroot_ref, o_ref):
        ssum = sp_ref[0] + sp_ref[1]
        cnt = cnt_ref[0, :, 0] + cnt_ref[1, :, 0]
        agg = ssum / jnp.maximum(cnt, 1.0)[:, None]
        y = (jnp.dot(agg, wrel_ref[...], preferred_element_type=jnp.float32)
             + b_ref[...]
             + jnp.dot(x_ref[...], wroot_ref[...],
                       preferred_element_type=jnp.float32))
        o_ref[...] = jnp.maximum(y, 0.0)

    return pl.pallas_call(
        body,
        grid=(n // bn,),
        in_specs=[
            pl.BlockSpec((_NC, bn, feat), lambda i: (0, i, 0)),
            pl.BlockSpec((_NC, bn, feat), lambda i: (0, i, 0)),
            pl.BlockSpec((bn, feat), lambda i: (i, 0)),
            pl.BlockSpec((feat, feat), lambda i: (0, 0)),
            pl.BlockSpec((1, feat), lambda i: (0, 0)),
            pl.BlockSpec((feat, feat), lambda i: (0, 0)),
        ],
        out_specs=pl.BlockSpec((bn, feat), lambda i: (i, 0)),
        out_shape=jax.ShapeDtypeStruct((n, feat), jnp.float32),
    )(sp, cntp, x, wrel, brel.reshape(1, feat), wroot)


def _tc_layer2(sp, cntp, h, wrel, brel, wroot, wd1a, wd1b, bd1):
    n, feat = h.shape
    dh = wd1a.shape[1]
    bn = 1000

    def body(sp_ref, cnt_ref, h_ref, wrel_ref, b_ref, wroot_ref,
             wa_ref, wb_ref, bd1_ref, a_ref):
        ssum = sp_ref[0] + sp_ref[1]
        cnt = cnt_ref[0, :, 0] + cnt_ref[1, :, 0]
        agg = ssum / jnp.maximum(cnt, 1.0)[:, None]
        z = (jnp.dot(agg, wrel_ref[...], preferred_element_type=jnp.float32)
             + b_ref[...]
             + jnp.dot(h_ref[...], wroot_ref[...],
                       preferred_element_type=jnp.float32))
        a = jnp.dot(z, wa_ref[...], preferred_element_type=jnp.float32)
        bo = (jnp.dot(z, wb_ref[...], preferred_element_type=jnp.float32)
              + bd1_ref[...])
        a_ref[...] = jnp.concatenate([a, bo], axis=1)

    return pl.pallas_call(
        body,
        grid=(n // bn,),
        in_specs=[
            pl.BlockSpec((_NC, bn, feat), lambda i: (0, i, 0)),
            pl.BlockSpec((_NC, bn, feat), lambda i: (0, i, 0)),
            pl.BlockSpec((bn, feat), lambda i: (i, 0)),
            pl.BlockSpec((feat, feat), lambda i: (0, 0)),
            pl.BlockSpec((1, feat), lambda i: (0, 0)),
            pl.BlockSpec((feat, feat), lambda i: (0, 0)),
            pl.BlockSpec((feat, dh), lambda i: (0, 0)),
            pl.BlockSpec((feat, dh), lambda i: (0, 0)),
            pl.BlockSpec((1, dh), lambda i: (0, 0)),
        ],
        out_specs=pl.BlockSpec((bn, 2 * dh), lambda i: (i, 0)),
        out_shape=jax.ShapeDtypeStruct((n, 2 * dh), jnp.float32),
    )(sp, cntp, h, wrel, brel.reshape(1, feat), wroot, wd1a, wd1b,
      bd1.reshape(1, dh))


def _decode_sc(ab, idx_s, idx_d, w, cvec, wd2, bd2):
    n, dh2 = ab.shape
    dh = dh2 // 2
    (e,) = idx_s.shape
    ept = e // _NW
    chunk = 80
    nchunk = ept // chunk
    nk = dh // _L

    bd2v = jnp.broadcast_to(bd2.reshape(1).astype(jnp.float32), (_L,))

    scr = [
        pltpu.VMEM((ept,), jnp.int32),
        pltpu.VMEM((ept,), jnp.int32),
        pltpu.VMEM((ept,), jnp.float32),
        pltpu.VMEM((chunk, dh2), jnp.float32),
        pltpu.VMEM((chunk, dh2), jnp.float32),
        pltpu.VMEM((dh,), jnp.float32),
        pltpu.VMEM((dh,), jnp.float32),
        pltpu.VMEM((_L,), jnp.float32),
        pltpu.VMEM((ept,), jnp.float32),
        pltpu.SemaphoreType.DMA,
        pltpu.SemaphoreType.DMA,
    ]

    @functools.partial(
        pl.kernel,
        out_type=jax.ShapeDtypeStruct((e,), jnp.float32),
        mesh=_vector_mesh(), scratch_types=scr,
        compiler_params=_sc_params())
    def k(ab_hbm, i0_hbm, i1_hbm, w_hbm, c_hbm, d_hbm, bd2_hbm,
          out_hbm, i0_v, i1_v, w_v, arows_v, brows_v, c_v, d_v, bd2_v,
          out_v, sem0, sem1):
        c = lax.axis_index("c")
        s = lax.axis_index("s")
        wid = c * _NS + s

        pltpu.sync_copy(c_hbm, c_v)
        pltpu.sync_copy(d_hbm, d_v)
        pltpu.sync_copy(bd2_hbm, bd2_v)
        pltpu.sync_copy(i0_hbm.at[pl.ds(wid * ept, ept)], i0_v)
        pltpu.sync_copy(i1_hbm.at[pl.ds(wid * ept, ept)], i1_v)
        pltpu.sync_copy(w_hbm.at[pl.ds(wid * ept, ept)], w_v)

        ck = [c_v[pl.ds(k * _L, _L)] for k in range(nk)]
        dk = [d_v[pl.ds(k * _L, _L)] for k in range(nk)]
        bd2r = bd2_v[...]
        lane = lax.iota(jnp.int32, 16)

        @pl.loop(0, nchunk)
        def _(i):
            cp0 = pltpu.async_copy(
                ab_hbm.at[i0_v.at[pl.ds(i * chunk, chunk)]], arows_v, sem0)
            cp1 = pltpu.async_copy(
                ab_hbm.at[i1_v.at[pl.ds(i * chunk, chunk)]], brows_v, sem1)
            cp0.wait()
            cp1.wait()

            @pl.loop(0, chunk // _L)
            def _(g):
                out_acc = jnp.zeros((_L,), jnp.float32)
                wrow = w_v[pl.ds(i * chunk + g * _L, _L)]
                for j in range(_L):
                    ws = jnp.full((_L,), wrow[j], jnp.float32)
                    arow = arows_v.at[g * _L + j]
                    brow = brows_v.at[g * _L + j]
                    acc = None
                    for k in range(nk):
                        t = (arow[pl.ds(k * _L, _L)]
                             + brow[pl.ds(dh + k * _L, _L)]
                             + ws * ck[k])
                        p = jnp.maximum(t, 0.0) * dk[k]
                        acc = p if acc is None else acc + p
                    tot = jnp.sum(acc)
                    out_acc = jnp.where(lane == j, tot, out_acc)
                out_v[pl.ds(i * chunk + g * _L, _L)] = out_acc + bd2r

        pltpu.sync_copy(out_v, out_hbm.at[pl.ds(wid * ept, ept)])

    return k(ab, idx_s, idx_d, w, cvec, wd2, bd2v)


def kernel(x, edge_index, edge_weight, edge_label_index, explicit_weight,
           W1rel, b1, W1root, W2rel, b2, W2root, Wd1, bd1, Wd2, bd2):
    src = edge_index[0]
    dst = edge_index[1]

    s1p, cntp = _segsum_sc(x, src, dst, edge_weight, with_cnt=True)
    h = _tc_layer1(s1p, cntp, x, W1rel, b1, W1root)
    s2p = _segsum_sc(h, src, dst, edge_weight, with_cnt=False)
    ab = _tc_layer2(s2p, cntp, h, W2rel, b2, W2root,
                    Wd1[:128], Wd1[128:256], bd1)
    out = _decode_sc(ab, edge_label_index[0], edge_label_index[1],
                     explicit_weight, Wd1[256], Wd2[:, 0], bd2)
    return out


# speedup vs baseline: 7.5922x; 1.5300x over previous
import dataclasses
import functools

import jax
import jax.numpy as jnp
from jax import lax
from jax.experimental import pallas as pl
from jax.experimental.pallas import tpu as pltpu
from jax.experimental.pallas import tpu_sc as plsc

_NC = 2
_NS = 16
_NW = _NC * _NS
_L = 16
_NPAD = 10240
_DBG_NO_LOOP = True
_DBG_NO_SPMEM_READ = True
_DBG_NO_HBM_WRITE = True
_DBG_NO_CNT = True


def _vector_mesh():
    return plsc.VectorSubcoreMesh(core_axis_name="c", subcore_axis_name="s")


def _sc_params(tc_tiling=True):
    cp = pltpu.CompilerParams()
    if "needs_layout_passes" in pltpu.CompilerParams.__dataclass_fields__:
        cp = dataclasses.replace(cp, needs_layout_passes=False)
    if not tc_tiling:
        cp = dataclasses.replace(cp, use_tc_tiling_on_sc=False)
    return cp


def _segsum_sc(x, src, dst, w, with_cnt):
    n, feat = x.shape
    (e,) = src.shape
    ept = e // _NW
    chunk = 80
    nsup = 5
    sub = ept // nsup
    nchunk = sub // chunk
    npad = _NPAD
    npart = npad // _NS

    outs = [jax.ShapeDtypeStruct((_NC * npad, feat), jnp.float32)]
    if with_cnt:
        outs.append(jax.ShapeDtypeStruct((_NC * npad, feat), jnp.float32))
    scr = [
        pltpu.VMEM((sub,), jnp.int32),
        pltpu.VMEM((chunk,), jnp.int32),
        pltpu.VMEM((chunk,), jnp.int32),
        pltpu.VMEM((sub,), jnp.float32),
        pltpu.VMEM((chunk, feat), jnp.float32),
        pltpu.VMEM((chunk, feat), jnp.float32),
        pltpu.VMEM((chunk, feat), jnp.float32),
        pltpu.VMEM_SHARED((npad, feat), jnp.float32),
        pltpu.SemaphoreType.DMA,
        pltpu.SemaphoreType.DMA,
        pltpu.SemaphoreType.DMA,
        pltpu.SemaphoreType.DMA,
    ]

    def body(refs):
        if with_cnt:
            (x_hbm, src_hbm, dst_hbm, w_hbm, acc_out, cnt_out,
             sidx_v, didx0, didx1, w_v, rows0, rows1, hop_v, acc_sh,
             sg0, sg1, sd0, sd1) = refs
        else:
            (x_hbm, src_hbm, dst_hbm, w_hbm, acc_out,
             sidx_v, didx0, didx1, w_v, rows0, rows1, hop_v, acc_sh,
             sg0, sg1, sd0, sd1) = refs
        c = lax.axis_index("c")
        s = lax.axis_index("s")
        wid = c * _NS + s
        didx = (didx0, didx1)
        rows = (rows0, rows1)
        sg = (sg0, sg1)
        sd = (sd0, sd1)

        zv = jnp.zeros((_L,), jnp.float32)
        ov = zv + 1.0

        def fill(buf, vec):
            for j in range(chunk):
                row = buf.at[j]
                for k in range(feat // _L):
                    row[pl.ds(k * _L, _L)] = vec

        def zero_acc():
            fill(hop_v, zv)
            for t in range(npart // chunk):
                pltpu.sync_copy(
                    hop_v, acc_sh.at[pl.ds(s * npart + t * chunk, chunk)])

        def write_acc(out_hbm):
            for t in range(npart // chunk):
                r0 = s * npart + t * chunk
                pltpu.sync_copy(acc_sh.at[pl.ds(r0, chunk)], hop_v)
                pltpu.sync_copy(hop_v,
                                out_hbm.at[pl.ds(c * npad + r0, chunk)])

        def g_start(q, b):
            pltpu.async_copy(
                x_hbm.at[sidx_v.at[pl.ds(q * chunk, chunk)]], rows[b], sg[b])

        def g_wait(q, b):
            pltpu.make_async_copy(
                x_hbm.at[sidx_v.at[pl.ds(q * chunk, chunk)]], rows[b],
                sg[b]).wait()

        def d_start(off, b):
            pltpu.async_copy(dst_hbm.at[pl.ds(off, chunk)], didx[b], sd[b])

        def d_wait(off, b):
            pltpu.make_async_copy(dst_hbm.at[pl.ds(off, chunk)], didx[b],
                                  sd[b]).wait()

        def scale_scatter(q, b):
            for g in range(chunk // _L):
                wrow = w_v[pl.ds(q * chunk + g * _L, _L)]
                for j in range(_L):
                    ws = jnp.full((_L,), wrow[j], jnp.float32)
                    row = rows[b].at[g * _L + j]
                    for k in range(feat // _L):
                        sl = pl.ds(k * _L, _L)
                        row[sl] = row[sl] * ws
            pltpu.sync_copy(rows[b], acc_sh.at[didx[b]], add=True)

        zero_acc()
        plsc.subcore_barrier()

        @pl.loop(0, nsup)
        def _(sc):
            base = wid * ept + sc * sub
            pltpu.sync_copy(src_hbm.at[pl.ds(base, sub)], sidx_v)
            pltpu.sync_copy(w_hbm.at[pl.ds(base, sub)], w_v)

            d_start(base, 0)
            g_start(0, 0)

            @pl.loop(0, (nchunk - 1) // 2)
            def _(ii):
                q0 = 2 * ii
                d_start(base + (q0 + 1) * chunk, 1)
                g_start(q0 + 1, 1)
                g_wait(q0, 0)
                d_wait(base + q0 * chunk, 0)
                scale_scatter(q0, 0)
                d_start(base + (q0 + 2) * chunk, 0)
                g_start(q0 + 2, 0)
                g_wait(q0 + 1, 1)
                d_wait(base + (q0 + 1) * chunk, 1)
                scale_scatter(q0 + 1, 1)

            qe = nchunk - 1
            g_wait(qe, 0)
            d_wait(base + qe * chunk, 0)
            scale_scatter(qe, 0)

        plsc.subcore_barrier()
        write_acc(acc_out)

        if with_cnt:
            plsc.subcore_barrier()
            zero_acc()
            fill(rows0, ov)
            plsc.subcore_barrier()
            nall = ept // chunk
            base = wid * ept
            d_start(base, 0)

            @pl.loop(0, (nall - 1) // 2)
            def _(ii):
                q0 = 2 * ii
                d_start(base + (q0 + 1) * chunk, 1)
                d_wait(base + q0 * chunk, 0)
                pltpu.sync_copy(rows0, acc_sh.at[didx[0]], add=True)
                d_start(base + (q0 + 2) * chunk, 0)
                d_wait(base + (q0 + 1) * chunk, 1)
                pltpu.sync_copy(rows0, acc_sh.at[didx[1]], add=True)

            qe = nall - 1
            d_wait(base + qe * chunk, 0)
            pltpu.sync_copy(rows0, acc_sh.at[didx[0]], add=True)

            plsc.subcore_barrier()
            write_acc(cnt_out)

    if with_cnt:
        @functools.partial(pl.kernel, out_type=tuple(outs),
                           mesh=_vector_mesh(), scratch_types=scr)
        def k(*refs):
            body(refs)

        o1, o2 = k(x, src, dst, w)
        return (o1.reshape(_NC, npad, feat), o2.reshape(_NC, npad, feat))
    else:

        @functools.partial(pl.kernel, out_type=outs[0],
                           mesh=_vector_mesh(), scratch_types=scr)
        def k(*refs):
            body(refs)

        return k(x, src, dst, w).reshape(_NC, npad, feat)


def _tc_layer1(sp, cntp, x, wrel, brel, wroot):
    n, feat = x.shape
    bn = 1000

    def body(sp_ref, cnt_ref, x_ref, wrel_ref, b_ref, wroot_ref, o_ref):
        ssum = sp_ref[0] + sp_ref[1]
        cnt = cnt_ref[0, :, 0] + cnt_ref[1, :, 0]
        agg = ssum / jnp.maximum(cnt, 1.0)[:, None]
        y = (jnp.dot(agg, wrel_ref[...], preferred_element_type=jnp.float32)
             + b_ref[...]
             + jnp.dot(x_ref[...], wroot_ref[...],
                       preferred_element_type=jnp.float32))
        o_ref[...] = jnp.maximum(y, 0.0)

    return pl.pallas_call(
        body,
        grid=(n // bn,),
        in_specs=[
            pl.BlockSpec((_NC, bn, feat), lambda i: (0, i, 0)),
            pl.BlockSpec((_NC, bn, feat), lambda i: (0, i, 0)),
            pl.BlockSpec((bn, feat), lambda i: (i, 0)),
            pl.BlockSpec((feat, feat), lambda i: (0, 0)),
            pl.BlockSpec((1, feat), lambda i: (0, 0)),
            pl.BlockSpec((feat, feat), lambda i: (0, 0)),
        ],
        out_specs=pl.BlockSpec((bn, feat), lambda i: (i, 0)),
        out_shape=jax.ShapeDtypeStruct((n, feat), jnp.float32),
    )(sp, cntp, x, wrel, brel.reshape(1, feat), wroot)


def _tc_layer2(sp, cntp, h, wrel, brel, wroot, wd1a, wd1b, bd1):
    n, feat = h.shape
    dh = wd1a.shape[1]
    bn = 1000

    def body(sp_ref, cnt_ref, h_ref, wrel_ref, b_ref, wroot_ref,
             wa_ref, wb_ref, bd1_ref, a_ref):
        ssum = sp_ref[0] + sp_ref[1]
        cnt = cnt_ref[0, :, 0] + cnt_ref[1, :, 0]
        agg = ssum / jnp.maximum(cnt, 1.0)[:, None]
        z = (jnp.dot(agg, wrel_ref[...], preferred_element_type=jnp.float32)
             + b_ref[...]
             + jnp.dot(h_ref[...], wroot_ref[...],
                       preferred_element_type=jnp.float32))
        a = jnp.dot(z, wa_ref[...], preferred_element_type=jnp.float32)
        bo = (jnp.dot(z, wb_ref[...], preferred_element_type=jnp.float32)
              + bd1_ref[...])
        a_ref[...] = jnp.concatenate([a, bo], axis=1)

    return pl.pallas_call(
        body,
        grid=(n // bn,),
        in_specs=[
            pl.BlockSpec((_NC, bn, feat), lambda i: (0, i, 0)),
            pl.BlockSpec((_NC, bn, feat), lambda i: (0, i, 0)),
            pl.BlockSpec((bn, feat), lambda i: (i, 0)),
            pl.BlockSpec((feat, feat), lambda i: (0, 0)),
            pl.BlockSpec((1, feat), lambda i: (0, 0)),
            pl.BlockSpec((feat, feat), lambda i: (0, 0)),
            pl.BlockSpec((feat, dh), lambda i: (0, 0)),
            pl.BlockSpec((feat, dh), lambda i: (0, 0)),
            pl.BlockSpec((1, dh), lambda i: (0, 0)),
        ],
        out_specs=pl.BlockSpec((bn, 2 * dh), lambda i: (i, 0)),
        out_shape=jax.ShapeDtypeStruct((n, 2 * dh), jnp.float32),
    )(sp, cntp, h, wrel, brel.reshape(1, feat), wroot, wd1a, wd1b,
      bd1.reshape(1, dh))


def _decode_sc(ab, idx_s, idx_d, w, cvec, wd2, bd2):
    n, dh2 = ab.shape
    dh = dh2 // 2
    (e,) = idx_s.shape
    ept = e // _NW
    chunk = 80
    nchunk = ept // chunk
    nk = dh // _L

    bd2v = jnp.broadcast_to(bd2.reshape(1).astype(jnp.float32), (_L,))

    scr = [
        pltpu.VMEM((ept,), jnp.int32),
        pltpu.VMEM((ept,), jnp.int32),
        pltpu.VMEM((ept,), jnp.float32),
        pltpu.VMEM((chunk, dh2), jnp.float32),
        pltpu.VMEM((chunk, dh2), jnp.float32),
        pltpu.VMEM((chunk, dh2), jnp.float32),
        pltpu.VMEM((chunk, dh2), jnp.float32),
        pltpu.VMEM((dh,), jnp.float32),
        pltpu.VMEM((dh,), jnp.float32),
        pltpu.VMEM((_L,), jnp.float32),
        pltpu.VMEM((ept,), jnp.float32),
        pltpu.SemaphoreType.DMA,
        pltpu.SemaphoreType.DMA,
        pltpu.SemaphoreType.DMA,
        pltpu.SemaphoreType.DMA,
    ]

    @functools.partial(
        pl.kernel,
        out_type=jax.ShapeDtypeStruct((e,), jnp.float32),
        mesh=_vector_mesh(), scratch_types=scr,
        compiler_params=_sc_params())
    def k(ab_hbm, i0_hbm, i1_hbm, w_hbm, c_hbm, d_hbm, bd2_hbm,
          out_hbm, i0_v, i1_v, w_v, ar0, ar1, br0, br1, c_v, d_v, bd2_v,
          out_v, sa0, sa1, sb0, sb1):
        c = lax.axis_index("c")
        s = lax.axis_index("s")
        wid = c * _NS + s
        ar = (ar0, ar1)
        br = (br0, br1)
        sa = (sa0, sa1)
        sb = (sb0, sb1)

        pltpu.sync_copy(c_hbm, c_v)
        pltpu.sync_copy(d_hbm, d_v)
        pltpu.sync_copy(bd2_hbm, bd2_v)
        pltpu.sync_copy(i0_hbm.at[pl.ds(wid * ept, ept)], i0_v)
        pltpu.sync_copy(i1_hbm.at[pl.ds(wid * ept, ept)], i1_v)
        pltpu.sync_copy(w_hbm.at[pl.ds(wid * ept, ept)], w_v)

        ck = [c_v[pl.ds(k * _L, _L)] for k in range(nk)]
        dk = [d_v[pl.ds(k * _L, _L)] for k in range(nk)]
        bd2r = bd2_v[...]
        lane = lax.iota(jnp.int32, 16)

        def g_start(q, b):
            pltpu.async_copy(
                ab_hbm.at[i0_v.at[pl.ds(q * chunk, chunk)]], ar[b], sa[b])
            pltpu.async_copy(
                ab_hbm.at[i1_v.at[pl.ds(q * chunk, chunk)]], br[b], sb[b])

        def g_wait(q, b):
            pltpu.make_async_copy(
                ab_hbm.at[i0_v.at[pl.ds(q * chunk, chunk)]], ar[b],
                sa[b]).wait()
            pltpu.make_async_copy(
                ab_hbm.at[i1_v.at[pl.ds(q * chunk, chunk)]], br[b],
                sb[b]).wait()

        def process(q, b):
            for g in range(chunk // _L):
                out_acc = jnp.zeros((_L,), jnp.float32)
                wrow = w_v[pl.ds(q * chunk + g * _L, _L)]
                for j in range(_L):
                    ws = jnp.full((_L,), wrow[j], jnp.float32)
                    arow = ar[b].at[g * _L + j]
                    brow = br[b].at[g * _L + j]
                    acc = None
                    for k in range(nk):
                        t = (arow[pl.ds(k * _L, _L)]
                             + brow[pl.ds(dh + k * _L, _L)]
                             + ws * ck[k])
                        p = jnp.maximum(t, 0.0) * dk[k]
                        acc = p if acc is None else acc + p
                    tot = jnp.sum(acc)
                    out_acc = jnp.where(lane == j, tot, out_acc)
                out_v[pl.ds(q * chunk + g * _L, _L)] = out_acc + bd2r

        g_start(0, 0)

        @pl.loop(0, (nchunk - 1) // 2)
        def _(ii):
            q0 = 2 * ii
            g_start(q0 + 1, 1)
            g_wait(q0, 0)
            process(q0, 0)
            g_start(q0 + 2, 0)
            g_wait(q0 + 1, 1)
            process(q0 + 1, 1)

        qe = nchunk - 1
        g_wait(qe, 0)
        process(qe, 0)

        pltpu.sync_copy(out_v, out_hbm.at[pl.ds(wid * ept, ept)])

    return k(ab, idx_s, idx_d, w, cvec, wd2, bd2v)


def kernel(x, edge_index, edge_weight, edge_label_index, explicit_weight,
           W1rel, b1, W1root, W2rel, b2, W2root, Wd1, bd1, Wd2, bd2):
    src = edge_index[0]
    dst = edge_index[1]

    s1p, cntp = _segsum_sc(x, src, dst, edge_weight, with_cnt=True)
    h = _tc_layer1(s1p, cntp, x, W1rel, b1, W1root)
    s2p = _segsum_sc(h, src, dst, edge_weight, with_cnt=False)
    ab = _tc_layer2(s2p, cntp, h, W2rel, b2, W2root,
                    Wd1[:128], Wd1[128:256], bd1)
    out = _decode_sc(ab, edge_label_index[0], edge_label_index[1],
                     explicit_weight, Wd1[256], Wd2[:, 0], bd2)
    return out
